# Initial kernel scaffold; baseline (speedup 1.0000x reference)
import jax
import jax.numpy as jnp
from jax.experimental import pallas as pl


def kernel(order, x, edge_index, edge_attr, batch, atom_emb, bond_emb_top, edge_lin_W, edge_lin_b, vn_emb, conv_bond_emb, conv_eps, conv_W1, conv_b1, conv_bn1_g, conv_bn1_b, conv_W2, conv_b2, bn_g, bn_b, vn_W1, vn_b1, vn_W2, vn_b2):
    raise NotImplementedError("write your pallas kernel here")


# speedup vs baseline: 1.9065x; 1.9065x over previous
import functools

import jax
import jax.numpy as jnp
from jax import lax
from jax.experimental import pallas as pl
from jax.experimental.pallas import tpu as pltpu
from jax.experimental.pallas import tpu_sc as plsc

F32 = jnp.float32
I32 = jnp.int32

G = 10
NP = 1000
N = G * NP
EPG = 16000
E = G * EPG
D = 128
NL = 3

EPG_PAD = 16384
E_PAD = 163840

_sc_params = pltpu.CompilerParams(needs_layout_passes=False)


def _tables_body(bond_ref, w_ref, b_ref, conv_ref, wtbl_ref, tbl_ref):
    cc = lax.broadcasted_iota(I32, (128, 1), 0)
    i5 = lax.broadcasted_iota(I32, (128, 5), 1)
    oh0 = (cc // 25 == i5).astype(F32)
    oh1 = ((cc // 5) % 5 == i5).astype(F32)
    oh2 = (cc % 5 == i5).astype(F32)
    dot = functools.partial(jnp.dot, preferred_element_type=F32,
                            precision=lax.Precision.HIGHEST)
    tbl3 = dot(oh0, bond_ref[0]) + dot(oh1, bond_ref[1]) + dot(oh2, bond_ref[2])
    logits = dot(tbl3, w_ref[...]) + b_ref[0]
    wtbl_ref[...] = jax.nn.sigmoid(logits)
    for l in range(NL):
        tbl_ref[l] = (dot(oh0, conv_ref[l, 0]) + dot(oh1, conv_ref[l, 1])
                      + dot(oh2, conv_ref[l, 2]))


_k_tables = pl.pallas_call(
    _tables_body,
    out_shape=[jax.ShapeDtypeStruct((128, 1), F32),
               jax.ShapeDtypeStruct((NL, 128, 128), F32)],
)


def _enc_body(x_ref, emb_ref, nf_ref):
    xv = x_ref[...]
    i10 = lax.broadcasted_iota(I32, (NP, 10), 1)
    acc = jnp.zeros((NP, D), F32)
    for f in range(9):
        oh = (xv[:, f:f + 1] == i10).astype(F32)
        acc = acc + jnp.dot(oh, emb_ref[f], preferred_element_type=F32,
                            precision=lax.Precision.HIGHEST)
    nf_ref[...] = acc


_k_enc = pl.pallas_call(
    _enc_body,
    grid=(G,),
    in_specs=[pl.BlockSpec((NP, 9), lambda g: (g, 0)),
              pl.BlockSpec((9, 10, D), lambda g: (0, 0, 0))],
    out_specs=pl.BlockSpec((NP, D), lambda g: (g, 0)),
    out_shape=jax.ShapeDtypeStruct((N, D), F32),
)


def _rows_body(code_ref, sub_ref, keep_ref, wtbl_ref, rows_ref):
    ii = lax.broadcasted_iota(I32, (128, 128), 0)
    jj = lax.broadcasted_iota(I32, (128, 128), 1)
    codes = code_ref[0]
    ohc = (jnp.broadcast_to(codes, (128, 128)) == ii).astype(F32)
    w_row = jnp.dot(wtbl_ref[...], ohc, preferred_element_type=F32,
                    precision=lax.Precision.HIGHEST)
    wk = w_row * keep_ref[0]
    diagwk = (ii == jj).astype(F32) * jnp.broadcast_to(wk, (128, 128))
    p = (jnp.broadcast_to(sub_ref[0], (128, 128)) == ii).astype(F32)
    rows_ref[...] = lax.dot_general(
        diagwk, p, (((1,), (1,)), ((), ())), preferred_element_type=F32,
        precision=lax.Precision.HIGHEST)


_k_rows = pl.pallas_call(
    _rows_body,
    grid=(G * EPG_PAD // 128,),
    in_specs=[pl.BlockSpec((1, 1, 128), lambda g: (g, 0, 0)),
              pl.BlockSpec((1, 1, 128), lambda g: (g, 0, 0)),
              pl.BlockSpec((1, 1, 128), lambda g: (g, 0, 0)),
              pl.BlockSpec((1, 128), lambda g: (0, 0))],
    out_specs=pl.BlockSpec((128, 128), lambda g: (g, 0)),
    out_shape=jax.ShapeDtypeStruct((G * EPG_PAD, 128), F32),
)


ADJ_ROWS = 7936


def _sc_adj_body(rowi_hbm, rows128_hbm, zer_hbm, adj_hbm, adj_sh, rowi_v,
                 rows_v):
    cid = lax.axis_index("c")
    sid = lax.axis_index("s")
    for gi in range(G // 2):
        g = cid * (G // 2) + gi
        pltpu.sync_copy(zer_hbm, adj_sh.at[pl.ds(sid * 496, 496)])
        plsc.subcore_barrier()
        tbase = g * EPG_PAD + sid * 1024

        def _chunk(ch, carry):
            base = tbase + ch * 128
            pltpu.sync_copy(rowi_hbm.at[pl.ds(base, 128)], rowi_v)
            pltpu.sync_copy(rows128_hbm.at[pl.ds(base, 128)], rows_v)
            pltpu.sync_copy(rows_v, adj_sh.at[rowi_v], add=True)
            return carry

        lax.fori_loop(0, 8, _chunk, 0)
        plsc.subcore_barrier()
        pltpu.sync_copy(adj_sh.at[pl.ds(sid * 496, 496)],
                        adj_hbm.at[g, pl.ds(sid * 496, 496)])
        plsc.subcore_barrier()


def _deg_body(adj_ref, rs_ref, cs_ref):
    adj = adj_ref[0]
    ii = lax.broadcasted_iota(I32, (NP, NP), 0)
    jj = lax.broadcasted_iota(I32, (NP, NP), 1)
    adj = adj + (ii == jj).astype(F32)
    rs_ref[0] = jnp.sum(adj, axis=1, keepdims=True)
    cs_ref[0] = jnp.sum(adj, axis=0, keepdims=True)


_k_deg = pl.pallas_call(
    _deg_body,
    grid=(G,),
    in_specs=[pl.BlockSpec((1, NP, NP), lambda g: (g, 0, 0))],
    out_specs=[pl.BlockSpec((1, NP, 1), lambda g: (g, 0, 0)),
               pl.BlockSpec((1, 1, NP), lambda g: (g, 0, 0))],
    out_shape=[jax.ShapeDtypeStruct((G, NP, 1), F32),
               jax.ShapeDtypeStruct((G, 1, NP), F32)],
)


def _stage1_body(adj_ref, nf_ref, rr_ref, rc_ref, y_ref):
    adj = adj_ref[0]
    ii = lax.broadcasted_iota(I32, (NP, NP), 0)
    jj = lax.broadcasted_iota(I32, (NP, NP), 1)
    adj = adj + (ii == jj).astype(F32)
    a = rr_ref[0] * adj * rc_ref[0]
    xp = nf_ref[...]
    y = xp
    for _ in range(3):
        xp = jnp.dot(a.astype(jnp.bfloat16), xp.astype(jnp.bfloat16),
                     preferred_element_type=F32)
        y = y + xp
    y_ref[...] = y


_k_stage1 = pl.pallas_call(
    _stage1_body,
    grid=(G,),
    in_specs=[pl.BlockSpec((1, NP, NP), lambda g: (g, 0, 0)),
              pl.BlockSpec((NP, D), lambda g: (g, 0)),
              pl.BlockSpec((1, NP, 1), lambda g: (g, 0, 0)),
              pl.BlockSpec((1, 1, NP), lambda g: (g, 0, 0))],
    out_specs=pl.BlockSpec((NP, D), lambda g: (g, 0)),
    out_shape=jax.ShapeDtypeStruct((N, D), F32),
)


AGG_ROWS = 10112


def _sc_edge_body(h_hbm, tbl_hbm, src_hbm, dst_hbm, code_hbm, zer_hbm,
                  agg_hbm, agg_sh, src_v, dst_v, code_v, hrows, erows,
                  sem1, sem2):
    cid = lax.axis_index("c")
    sid = lax.axis_index("s")
    pltpu.sync_copy(zer_hbm, agg_sh.at[pl.ds(sid * 632, 632)])
    plsc.subcore_barrier()
    ebase = (cid * 16 + sid) * 5120

    def _chunk(ch, carry):
        base = ebase + ch * 128
        pltpu.sync_copy(src_hbm.at[pl.ds(base, 128)], src_v)
        pltpu.sync_copy(dst_hbm.at[pl.ds(base, 128)], dst_v)
        pltpu.sync_copy(code_hbm.at[pl.ds(base, 128)], code_v)
        pltpu.async_copy(h_hbm.at[src_v], hrows, sem1).wait()
        pltpu.async_copy(tbl_hbm.at[code_v], erows, sem2).wait()

        def _edge(e, c2):
            mfac = jnp.where(base + e < E, 1.0, 0.0).astype(F32)
            for c in range(8):
                hseg = hrows[e, pl.ds(c * 16, 16)]
                eseg = erows[e, pl.ds(c * 16, 16)]
                hrows[e, pl.ds(c * 16, 16)] = jnp.maximum(hseg + eseg, 0.0) * mfac
            return c2

        lax.fori_loop(0, 128, _edge, 0)
        pltpu.sync_copy(hrows, agg_sh.at[dst_v], add=True)
        return carry

    lax.fori_loop(0, 40, _chunk, 0)
    plsc.subcore_barrier()
    pltpu.sync_copy(agg_sh.at[pl.ds(sid * 632, 632)],
                    agg_hbm.at[cid, pl.ds(sid * 632, 632)])


@functools.cache
def _sc_kernels():
    mesh = plsc.VectorSubcoreMesh(core_axis_name="c", subcore_axis_name="s")
    sc_adj = pl.kernel(
        _sc_adj_body,
        out_type=jax.ShapeDtypeStruct((G, ADJ_ROWS, 128), F32),
        mesh=mesh,
        compiler_params=_sc_params,
        scratch_types=[
            pltpu.VMEM_SHARED((ADJ_ROWS, 128), F32),
            pltpu.VMEM((128,), I32),
            pltpu.VMEM((128, 128), F32),
        ],
    )
    sc_edge = pl.kernel(
        _sc_edge_body,
        out_type=jax.ShapeDtypeStruct((2, AGG_ROWS, D), F32),
        mesh=mesh,
        compiler_params=_sc_params,
        scratch_types=[
            pltpu.VMEM_SHARED((AGG_ROWS, D), F32),
            pltpu.VMEM((128,), I32),
            pltpu.VMEM((128,), I32),
            pltpu.VMEM((128,), I32),
            pltpu.VMEM((128, D), F32),
            pltpu.VMEM((128, D), F32),
            pltpu.SemaphoreType.DMA,
            pltpu.SemaphoreType.DMA,
        ],
    )
    return sc_adj, sc_edge


def _layer_body(last, h_ref, a0_ref, a1_ref, eps_ref, w1_ref, b1_ref, g1_ref,
                bb1_ref, w2_ref, b2_ref, g2_ref, bb2_ref, vn_ref, vw1_ref,
                vb1_ref, vw2_ref, vb2_ref, hn_ref, hi_ref, vnn_ref):
    h_in = h_ref[...]
    z0 = (1.0 + eps_ref[0, 0]) * h_in + a0_ref[...] + a1_ref[...]
    z = jnp.dot(z0.astype(jnp.bfloat16), w1_ref[...].astype(jnp.bfloat16),
                preferred_element_type=F32) + b1_ref[...]
    m = jnp.mean(z, axis=0, keepdims=True)
    v = jnp.mean((z - m) ** 2, axis=0, keepdims=True)
    z = (z - m) / jnp.sqrt(v + 1e-5) * g1_ref[...] + bb1_ref[...]
    z = jnp.maximum(z, 0.0)
    z2 = jnp.dot(z.astype(jnp.bfloat16), w2_ref[...].astype(jnp.bfloat16),
                 preferred_element_type=F32) + b2_ref[...]
    m2 = jnp.mean(z2, axis=0, keepdims=True)
    v2 = jnp.mean((z2 - m2) ** 2, axis=0, keepdims=True)
    hn = (z2 - m2) / jnp.sqrt(v2 + 1e-5) * g2_ref[...] + bb2_ref[...]
    hn_ref[...] = hn
    if last:
        hi_ref[...] = hn
        vnn_ref[...] = vn_ref[...]
    else:
        hact = jnp.maximum(hn, 0.0)
        pooled = jnp.sum(h_in.reshape(G, NP, D), axis=1)
        vtmp = pooled + vn_ref[...]
        t = jnp.maximum(
            jnp.dot(vtmp.astype(jnp.bfloat16),
                    vw1_ref[...].astype(jnp.bfloat16),
                    preferred_element_type=F32)
            + vb1_ref[...], 0.0)
        vnn = jnp.maximum(
            jnp.dot(t.astype(jnp.bfloat16),
                    vw2_ref[...].astype(jnp.bfloat16),
                    preferred_element_type=F32)
            + vb2_ref[...], 0.0)
        vnn_ref[...] = vnn
        hi_ref[...] = (hact.reshape(G, NP, D) + vnn[:, None, :]).reshape(N, D)


_k_layer = [
    pl.pallas_call(
        functools.partial(_layer_body, l == NL - 1),
        out_shape=[jax.ShapeDtypeStruct((N, D), F32),
                   jax.ShapeDtypeStruct((N, D), F32),
                   jax.ShapeDtypeStruct((G, D), F32)],
    )
    for l in range(NL)
]


def kernel(order, x, edge_index, edge_attr, batch, atom_emb, bond_emb_top,
           edge_lin_W, edge_lin_b, vn_emb, conv_bond_emb, conv_eps, conv_W1,
           conv_b1, conv_bn1_g, conv_bn1_b, conv_W2, conv_b2, bn_g, bn_b,
           vn_W1, vn_b1, vn_W2, vn_b2):
    src = edge_index[0].astype(I32)
    dst = edge_index[1].astype(I32)
    ea = edge_attr.astype(I32)
    code = ea[:, 0] * 25 + ea[:, 1] * 5 + ea[:, 2]

    lin = (src % NP) * NP + (dst % NP)
    cell = (src // NP) * (NP * NP) + lin
    sort_idx = jnp.argsort(cell)
    sorted_cell = cell[sort_idx]
    first = jnp.concatenate(
        [jnp.ones((1,), bool), sorted_cell[1:] != sorted_cell[:-1]])
    keep = first[jnp.argsort(sort_idx)].astype(F32)

    def pad_pg(a):
        a2 = a.reshape(G, EPG)
        a2 = jnp.pad(a2, ((0, 0), (0, EPG_PAD - EPG)))
        return a2.reshape(G * EPG_PAD)

    rowi_p = pad_pg(lin // 128)
    sub_p = pad_pg(lin % 128).reshape(-1, 1, 128)
    code_p = pad_pg(code).reshape(-1, 1, 128)
    keep_p = pad_pg(keep).reshape(-1, 1, 128)

    wtbl2, tblpad = _k_tables(bond_emb_top, edge_lin_W, edge_lin_b,
                              conv_bond_emb)
    nf = _k_enc(x.astype(I32), atom_emb)

    sc_adj, sc_edge = _sc_kernels()
    rows128 = _k_rows(code_p, sub_p, keep_p, wtbl2.reshape(1, 128))
    zer1 = jnp.zeros((496, 128), F32)
    adj = sc_adj(rowi_p, rows128, zer1)
    adj = adj.reshape(G, ADJ_ROWS * 128)[:, :NP * NP].reshape(G, NP, NP)
    rowsum, colsum = _k_deg(adj)
    rr = jnp.power(colsum, -0.5).reshape(G, NP, 1)
    rc = jnp.power(rowsum, -0.5).reshape(G, 1, NP)
    ysum = _k_stage1(adj, nf, rr, rc)
    h_in = ysum * (1.0 / (order + 1.0)) + vn_emb[0][None, :]

    src_r = jnp.pad(src, (0, E_PAD - E))
    dst_r = jnp.pad(dst, (0, E_PAD - E))
    code_r = jnp.pad(code, (0, E_PAD - E))
    zer2 = jnp.zeros((632, D), F32)
    vn = jnp.broadcast_to(vn_emb[0], (G, D))

    h_new = None
    for l in range(NL):
        aggp = sc_edge(h_in, tblpad[l], src_r, dst_r, code_r, zer2)
        lv = min(l, NL - 2)
        h_new, h_in, vn = _k_layer[l](
            h_in, aggp[0, :N], aggp[1, :N], conv_eps[l].reshape(1, 1),
            conv_W1[l], conv_b1[l].reshape(1, -1),
            conv_bn1_g[l].reshape(1, -1), conv_bn1_b[l].reshape(1, -1),
            conv_W2[l], conv_b2[l].reshape(1, -1),
            bn_g[l].reshape(1, -1), bn_b[l].reshape(1, -1),
            vn, vn_W1[lv], vn_b1[lv].reshape(1, -1),
            vn_W2[lv], vn_b2[lv].reshape(1, -1))
    return h_new
